# Initial kernel scaffold; baseline (speedup 1.0000x reference)
import jax
import jax.numpy as jnp
from jax.experimental import pallas as pl


def kernel(x, labels):
    raise NotImplementedError("write your pallas kernel here")


# speedup vs baseline: 50.0068x; 50.0068x over previous
import jax
import jax.numpy as jnp
from jax.experimental import pallas as pl
from jax.experimental.pallas import tpu as pltpu

_NBINS = 51
_BW = 2.0 / (_NBINS - 1)
_INV_BW = 1.0 / _BW
_BS = 1024
_D = 128
_ROWS_PER_STEP = 128
_N_CORES = 2
_STEPS = _BS // _ROWS_PER_STEP // _N_CORES
_ACC_ROWS = 56


def _hist_body(x_row_ref, xt_ref, lab_row_ref, lab_col_ref, pos_ref, all_ref):
    c = pl.program_id(0)
    j = pl.program_id(1)

    @pl.when(j == 0)
    def _init():
        pos_ref[...] = jnp.zeros_like(pos_ref)
        all_ref[...] = jnp.zeros_like(all_ref)

    r = _ROWS_PER_STEP
    s = jnp.dot(x_row_ref[...], xt_ref[...], preferred_element_type=jnp.float32)

    b = jnp.floor((s + 1.0) * _INV_BW).astype(jnp.int32)
    v = b.astype(jnp.float32) * _BW - 1.0
    wl = (v + _BW - s) * _INV_BW
    wh = (s - v) * _INV_BW
    bh = jnp.clip(b + 1, 0, _NBINS - 1)

    row0 = (c * _STEPS + j) * r
    rows = row0 + jax.lax.broadcasted_iota(jnp.int32, (r, _BS), 0)
    cols = jax.lax.broadcasted_iota(jnp.int32, (r, _BS), 1)
    valid = rows < cols
    eq = lab_row_ref[...] == lab_col_ref[...]
    mpos = valid & eq

    zero = jnp.zeros_like(s)
    p_lo = jnp.where(mpos, wl, zero)
    p_hi = jnp.where(mpos, wh, zero)
    a_lo = jnp.where(valid, wl, zero)
    a_hi = jnp.where(valid, wh, zero)

    for k in range(_NBINS):
        m_lo = b == k
        m_hi = bh == k
        cp = jnp.where(m_lo, p_lo, zero) + jnp.where(m_hi, p_hi, zero)
        ca = jnp.where(m_lo, a_lo, zero) + jnp.where(m_hi, a_hi, zero)
        pos_ref[0, k, :] = pos_ref[0, k, :] + jnp.sum(cp, axis=0)
        all_ref[0, k, :] = all_ref[0, k, :] + jnp.sum(ca, axis=0)

    pos_ref[0, _NBINS, :] = pos_ref[0, _NBINS, :] + jnp.sum(
        mpos.astype(jnp.float32), axis=0)
    all_ref[0, _NBINS, :] = all_ref[0, _NBINS, :] + jnp.sum(
        valid.astype(jnp.float32), axis=0)


def _finalize_body(pos_ref, all_ref, out_ref):
    hp2 = pos_ref[0] + pos_ref[1]
    ha2 = all_ref[0] + all_ref[1]
    ones = jnp.ones((1, _BS), jnp.float32)
    dn = (((1,), (1,)), ((), ()))
    hp = jax.lax.dot_general(ones, hp2, dn,
                             preferred_element_type=jnp.float32)
    ha = jax.lax.dot_general(ones, ha2, dn,
                             preferred_element_type=jnp.float32)

    npos = hp[0:1, _NBINS:_NBINS + 1]
    nall = ha[0:1, _NBINS:_NBINS + 1]
    nneg = nall - npos

    lane = jax.lax.broadcasted_iota(jnp.int32, (1, _ACC_ROWS), 1)
    bin_mask = lane < _NBINS
    hp_b = jnp.where(bin_mask, hp, 0.0)
    hn_b = jnp.where(bin_mask, ha - hp, 0.0)
    hist_pos = hp_b / npos
    hist_neg = hn_b / nneg

    m_i = jax.lax.broadcasted_iota(jnp.int32, (_ACC_ROWS, _ACC_ROWS), 0)
    k_i = jax.lax.broadcasted_iota(jnp.int32, (_ACC_ROWS, _ACC_ROWS), 1)
    tri = (m_i <= k_i).astype(jnp.float32)
    cdf = jnp.dot(hist_pos, tri, preferred_element_type=jnp.float32)

    out_ref[...] = jnp.sum(hist_neg * cdf, axis=1, keepdims=True)


def kernel(x, labels):
    lab = labels.astype(jnp.int32)
    lab_row = lab.reshape(_BS, 1)
    lab_col = lab.reshape(1, _BS)
    xt = x.T

    acc_shape = (_N_CORES, _ACC_ROWS, _BS)
    pos_acc, all_acc = pl.pallas_call(
        _hist_body,
        grid=(_N_CORES, _STEPS),
        in_specs=[
            pl.BlockSpec((_ROWS_PER_STEP, _D),
                         lambda c, j: (c * _STEPS + j, 0)),
            pl.BlockSpec((_D, _BS), lambda c, j: (0, 0)),
            pl.BlockSpec((_ROWS_PER_STEP, 1),
                         lambda c, j: (c * _STEPS + j, 0)),
            pl.BlockSpec((1, _BS), lambda c, j: (0, 0)),
        ],
        out_specs=[
            pl.BlockSpec((1, _ACC_ROWS, _BS), lambda c, j: (c, 0, 0)),
            pl.BlockSpec((1, _ACC_ROWS, _BS), lambda c, j: (c, 0, 0)),
        ],
        out_shape=[
            jax.ShapeDtypeStruct(acc_shape, jnp.float32),
            jax.ShapeDtypeStruct(acc_shape, jnp.float32),
        ],
        compiler_params=pltpu.CompilerParams(
            dimension_semantics=("parallel", "arbitrary")),
    )(x, xt, lab_row, lab_col)

    loss = pl.pallas_call(
        _finalize_body,
        out_shape=jax.ShapeDtypeStruct((1, 1), jnp.float32),
    )(pos_acc, all_acc)
    return loss[0, 0]


# speedup vs baseline: 218.8974x; 4.3774x over previous
import jax
import jax.numpy as jnp
from jax.experimental import pallas as pl
from jax.experimental.pallas import tpu as pltpu

_NBINS = 51
_BW = 2.0 / (_NBINS - 1)
_INV_BW = 1.0 / _BW
_BS = 1024
_D = 128
_T = 128
_NT = _BS // _T
_NPAIRS = _NT * (_NT + 1) // 2
_NTHRESH = _NBINS + 2
_CNT_ROW = 54
_ACC_ROWS = 56


def _decode(t):
    rb = (
        (t >= 8).astype(jnp.int32) + (t >= 15) + (t >= 21) + (t >= 26)
        + (t >= 30) + (t >= 33) + (t >= 35)
    )
    cb = t - (8 * rb - (rb * (rb - 1)) // 2) + rb
    return rb, cb


def _tree8(a):
    return jnp.sum(a.reshape(16, 8, 128), axis=0)


def _sweep(sp, evm, vm, rp_ref, ra_ref):
    for ti in range(_NTHRESH):
        r = jnp.maximum(sp - (ti - 1.0), 0.0)
        ra = r if vm is None else r * vm
        rp = r * evm
        ra_ref[ti] = ra_ref[ti] + _tree8(ra)
        rp_ref[ti] = rp_ref[ti] + _tree8(rp)
    if vm is None:
        ra_ref[_CNT_ROW] = ra_ref[_CNT_ROW] + 16.0
    else:
        ra_ref[_CNT_ROW] = ra_ref[_CNT_ROW] + _tree8(vm)
    rp_ref[_CNT_ROW] = rp_ref[_CNT_ROW] + _tree8(evm)


def _hist_body(xr_ref, xc_ref, lr_ref, lc_ref, rp_ref, ra_ref):
    t = pl.program_id(0)
    rb, cb = _decode(t)

    @pl.when(t == 0)
    def _init():
        rp_ref[...] = jnp.zeros_like(rp_ref)
        ra_ref[...] = jnp.zeros_like(ra_ref)

    dn = (((1,), (1,)), ((), ()))
    s = jax.lax.dot_general(xr_ref[...], xc_ref[...], dn,
                            preferred_element_type=jnp.float32)
    sp = s * _INV_BW + _INV_BW
    eq = lr_ref[...] == lc_ref[0]

    @pl.when(rb == cb)
    def _diag():
        ii = jax.lax.broadcasted_iota(jnp.int32, (_T, _T), 0)
        jj = jax.lax.broadcasted_iota(jnp.int32, (_T, _T), 1)
        vm = jnp.where(ii == jj, 0.0, 0.5)
        evm = jnp.where(eq, vm, 0.0)
        _sweep(sp, evm, vm, rp_ref, ra_ref)

    @pl.when(rb < cb)
    def _offdiag():
        evm = jnp.where(eq, 1.0, 0.0)
        _sweep(sp, evm, None, rp_ref, ra_ref)


def _finalize_body(rp_ref, ra_ref, out_ref):
    def hist_rows(ref):
        h = []
        for k in range(_NBINS):
            h.append(ref[k] - 2.0 * ref[k + 1] + ref[k + 2])
        h.append(ref[_CNT_ROW])
        return jnp.concatenate([r.reshape(1, 8, 128) for r in h], axis=0)

    hp3 = hist_rows(rp_ref)
    ha3 = hist_rows(ra_ref)
    hp2 = jnp.sum(hp3, axis=1)
    ha2 = jnp.sum(ha3, axis=1)
    ones = jnp.ones((1, 128), jnp.float32)
    dn = (((1,), (1,)), ((), ()))
    hp = jax.lax.dot_general(ones, hp2, dn,
                             preferred_element_type=jnp.float32)
    ha = jax.lax.dot_general(ones, ha2, dn,
                             preferred_element_type=jnp.float32)

    npos = hp[0:1, _NBINS:_NBINS + 1]
    nall = ha[0:1, _NBINS:_NBINS + 1]
    nneg = nall - npos

    lane = jax.lax.broadcasted_iota(jnp.int32, (1, 52), 1)
    bin_mask = lane < _NBINS
    hp_b = jnp.where(bin_mask, hp, 0.0)
    hn_b = jnp.where(bin_mask, ha - hp, 0.0)
    hist_pos = hp_b / npos
    hist_neg = hn_b / nneg

    m_i = jax.lax.broadcasted_iota(jnp.int32, (52, 52), 0)
    k_i = jax.lax.broadcasted_iota(jnp.int32, (52, 52), 1)
    tri = (m_i <= k_i).astype(jnp.float32)
    cdf = jnp.dot(hist_pos, tri, preferred_element_type=jnp.float32)

    out_ref[...] = jnp.sum(hist_neg * cdf, axis=1, keepdims=True)


def kernel(x, labels):
    lab = labels.astype(jnp.int32)
    lab_row = lab.reshape(_BS, 1)
    lab_col = lab.reshape(_NT, 1, _T)

    acc_shape = (_ACC_ROWS, 8, _T)
    rp_acc, ra_acc = pl.pallas_call(
        _hist_body,
        grid=(_NPAIRS,),
        in_specs=[
            pl.BlockSpec((_T, _D), lambda t: (_decode(t)[0], 0)),
            pl.BlockSpec((_T, _D), lambda t: (_decode(t)[1], 0)),
            pl.BlockSpec((_T, 1), lambda t: (_decode(t)[0], 0)),
            pl.BlockSpec((1, 1, _T), lambda t: (_decode(t)[1], 0, 0)),
        ],
        out_specs=[
            pl.BlockSpec(acc_shape, lambda t: (0, 0, 0)),
            pl.BlockSpec(acc_shape, lambda t: (0, 0, 0)),
        ],
        out_shape=[
            jax.ShapeDtypeStruct(acc_shape, jnp.float32),
            jax.ShapeDtypeStruct(acc_shape, jnp.float32),
        ],
        compiler_params=pltpu.CompilerParams(
            dimension_semantics=("arbitrary",)),
    )(x, x, lab_row, lab_col)

    loss = pl.pallas_call(
        _finalize_body,
        out_shape=jax.ShapeDtypeStruct((1, 1), jnp.float32),
    )(rp_acc, ra_acc)
    return loss[0, 0]
